# baseline (speedup 1.0000x reference)
import functools

import jax
import jax.numpy as jnp
from jax import lax
from jax.experimental import pallas as pl
from jax.experimental.pallas import tpu as pltpu
from jax.experimental.pallas import tpu_sc as plsc

B = 4096
F = 26
D = 16
NW = 32
EPW = B // NW
RPW = EPW * F
PACK = 128 // D
OUT_ROWS = B * D // 128
ORPW = OUT_ROWS // NW


@functools.cache
def _make_sc_pool():
    mesh = plsc.VectorSubcoreMesh(core_axis_name="c", subcore_axis_name="s")

    @functools.partial(
        pl.kernel,
        mesh=mesh,
        compiler_params=pltpu.CompilerParams(use_tc_tiling_on_sc=False,
                                             needs_layout_passes=False),
        out_type=jax.ShapeDtypeStruct((OUT_ROWS, 128), jnp.float32),
        scratch_types=[
            pltpu.VMEM((EPW, F), jnp.int32),
            pltpu.VMEM((EPW, F), jnp.float32),
            pltpu.VMEM((RPW, D), jnp.float32),
            pltpu.VMEM((ORPW, 128), jnp.float32),
            pltpu.SemaphoreType.DMA,
        ],
    )
    def sc_pool(idx_hbm, fv_hbm, table_hbm, out_hbm,
                idx_v, fv_v, rows_v, bi_v, sem):
        wid = lax.axis_index("s") * 2 + lax.axis_index("c")
        base = wid * EPW
        pltpu.sync_copy(idx_hbm.at[pl.ds(base, EPW)], idx_v)
        pltpu.sync_copy(fv_hbm.at[pl.ds(base, EPW)], fv_v)

        def fire(e, carry):
            pltpu.async_copy(table_hbm.at[idx_v.at[e]],
                             rows_v.at[pl.ds(e * F, F)], sem)
            return carry
        lax.fori_loop(0, EPW, fire, 0)
        pltpu.make_async_copy(table_hbm.at[pl.ds(0, RPW)], rows_v, sem).wait()

        def e_body(e, carry):
            s = jnp.zeros((D,), jnp.float32)
            q = jnp.zeros((D,), jnp.float32)
            for f in range(F):
                row = rows_v[e * F + f]
                vb = plsc.load_gather(
                    fv_v, [jnp.full((D,), e, jnp.int32),
                           jnp.full((D,), f, jnp.int32)])
                ve = row * vb
                s = s + ve
                q = q + ve * ve
            bi = (s * s - q) * 0.5
            bi_v[e // PACK, pl.ds((e % PACK) * D, D)] = bi
            return carry

        lax.fori_loop(0, EPW, e_body, 0)
        pltpu.sync_copy(bi_v, out_hbm.at[pl.ds(wid * ORPW, ORPW)])

    return sc_pool


def _tc_mlp(bi_p, W1, b1, W2, b2, W3, b3):
    eye = jnp.eye(PACK, dtype=jnp.float32)
    W1p = jnp.kron(eye, W1)
    W2p = jnp.kron(eye, W2)
    W3p = jnp.kron(eye, W3)
    b1p = jnp.tile(b1, PACK).reshape(1, -1)
    b2p = jnp.tile(b2, PACK).reshape(1, -1)
    b3p = jnp.tile(b3, PACK).reshape(1, -1)

    def body(bi_ref, W1_ref, b1_ref, W2_ref, b2_ref, W3_ref, b3_ref, out_ref):
        h = jnp.maximum(
            jnp.dot(bi_ref[...], W1_ref[...], preferred_element_type=jnp.float32)
            + b1_ref[...], 0.0)
        h = jnp.maximum(
            jnp.dot(h, W2_ref[...], preferred_element_type=jnp.float32)
            + b2_ref[...], 0.0)
        o = jnp.dot(h, W3_ref[...], preferred_element_type=jnp.float32) + b3_ref[...]
        out_ref[...] = jax.nn.sigmoid(o)

    out = pl.pallas_call(
        body,
        out_shape=jax.ShapeDtypeStruct((OUT_ROWS, PACK), jnp.float32),
    )(bi_p, W1p, b1p, W2p, b2p, W3p, b3p)
    return out.reshape(B, 1)


def kernel(feat_index, feat_value, emb_table, W1, b1, W2, b2, W3, b3):
    fidx = feat_index.astype(jnp.int32)
    bi_p = _make_sc_pool()(fidx, feat_value, emb_table)
    return _tc_mlp(bi_p, W1, b1, W2, b2, W3, b3)
